# baseline (speedup 1.0000x reference)
import functools

import jax
import jax.numpy as jnp
import numpy as np
from jax import lax
from jax.experimental import pallas as pl
from jax.experimental.pallas import tpu as pltpu
from jax.experimental.pallas import tpu_sc as plsc

_OH_SIZES = (64, 48, 32, 32, 16, 16, 16, 8, 8, 8, 8, 8, 8, 8, 8, 8)
_NCOL = int(np.sum(_OH_SIZES))
_NFIELD = len(_OH_SIZES)
_LANES = 16
_PERIOD = 2 * _NCOL
_CHUNKS = _PERIOD // _LANES


def _build_combo_table() -> np.ndarray:
    col2field = np.repeat(np.arange(_NFIELD, dtype=np.int32),
                          np.asarray(_OH_SIZES))
    col2local = np.concatenate(
        [np.arange(s, dtype=np.int32) for s in _OH_SIZES])
    q = np.arange(_PERIOD, dtype=np.int32)
    r, c = q // _NCOL, q % _NCOL
    fidx = col2field[c] + _NFIELD * r
    lidx = col2local[c]
    return (fidx | (lidx << 16)).astype(np.int32)


_COMBO_NP = _build_combo_table()


@functools.lru_cache(maxsize=None)
def _make_sc_call(batch: int):
    info = plsc.get_sparse_core_info()
    nc, ns = info.num_cores, info.num_subcores
    nw = nc * ns
    rows_w = batch // nw
    sub_rows = 64
    nsb = rows_w // sub_rows
    pairs = sub_rows // 2
    buf_words = sub_rows * _NCOL
    mesh = plsc.VectorSubcoreMesh(core_axis_name="c", subcore_axis_name="s")

    @functools.partial(
        pl.kernel,
        mesh=mesh,
        compiler_params=pltpu.CompilerParams(needs_layout_passes=False),
        out_type=jax.ShapeDtypeStruct((batch * _NCOL,), jnp.float32),
        scratch_types=[
            pltpu.VMEM((rows_w * _NFIELD,), jnp.int32),
            pltpu.VMEM((_PERIOD,), jnp.int32),
            pltpu.VMEM((buf_words,), jnp.float32),
            pltpu.VMEM((buf_words,), jnp.float32),
            pltpu.SemaphoreType.DMA,
            pltpu.SemaphoreType.DMA,
        ],
    )
    def sc_call(data_hbm, combo_hbm, out_hbm, data_v, combo_v,
                buf0, buf1, sem0, sem1):
        wid = lax.axis_index("s") * nc + lax.axis_index("c")
        row0 = wid * rows_w
        pltpu.sync_copy(data_hbm.at[pl.ds(row0 * _NFIELD, rows_w * _NFIELD)],
                        data_v)
        pltpu.sync_copy(combo_hbm, combo_v)

        bufs, sems = (buf0, buf1), (sem0, sem1)
        inflight = [None, None]
        for sb in range(nsb):
            buf, sem = bufs[sb % 2], sems[sb % 2]
            if inflight[sb % 2] is not None:
                inflight[sb % 2].wait()
            for j in range(_CHUNKS):
                combo_j = combo_v[pl.ds(j * _LANES, _LANES)]
                f_j = combo_j & 0xFFFF
                l_j = combo_j >> 16

                def pair_body(lp, carry, f_j=f_j, l_j=l_j, j=j, buf=buf):
                    base = (sb * pairs + lp) * (2 * _NFIELD)
                    thr = plsc.load_gather(data_v, [f_j + base])
                    vals = jnp.where(l_j <= thr,
                                     jnp.float32(1.0), jnp.float32(0.0))
                    buf[pl.ds(lp * _PERIOD + j * _LANES, _LANES)] = vals
                    return carry

                lax.fori_loop(0, pairs, pair_body, 0)
            dst = out_hbm.at[pl.ds((row0 + sb * sub_rows) * _NCOL, buf_words)]
            inflight[sb % 2] = pltpu.async_copy(buf, dst, sem)
        inflight[0].wait()
        inflight[1].wait()

    return sc_call


def kernel(data, accum_mat):
    del accum_mat
    batch = data.shape[0]
    combo = jnp.asarray(_COMBO_NP)
    out_flat = _make_sc_call(batch)(data.reshape(-1).astype(jnp.int32), combo)
    return out_flat.reshape(batch, _NCOL)


# speedup vs baseline: 2.1405x; 2.1405x over previous
import functools

import jax
import jax.numpy as jnp
import numpy as np
from jax import lax
from jax.experimental import pallas as pl
from jax.experimental.pallas import tpu as pltpu
from jax.experimental.pallas import tpu_sc as plsc

_OH_SIZES = (64, 48, 32, 32, 16, 16, 16, 8, 8, 8, 8, 8, 8, 8, 8, 8)
_NCOL = int(np.sum(_OH_SIZES))
_NFIELD = len(_OH_SIZES)
_LANES = 16
_CHUNKS = 19
_CHUNK_STARTS = tuple(list(range(0, 288, 16)) + [_NCOL - 16])


def _build_combo_table() -> np.ndarray:
    col2field = np.repeat(np.arange(_NFIELD, dtype=np.int32),
                          np.asarray(_OH_SIZES))
    col2local = np.concatenate(
        [np.arange(s, dtype=np.int32) for s in _OH_SIZES])
    cols = np.concatenate(
        [np.arange(s, s + _LANES) for s in _CHUNK_STARTS])
    fidx = col2field[cols]
    lidx = col2local[cols]
    return (fidx | (lidx << 16)).astype(np.int32)


_COMBO_NP = _build_combo_table()


@functools.lru_cache(maxsize=None)
def _make_sc_call(batch: int):
    info = plsc.get_sparse_core_info()
    nc, ns = info.num_cores, info.num_subcores
    nw = nc * ns
    rows_w = batch // nw
    sub_rows = 64
    nsb = rows_w // sub_rows
    mesh = plsc.VectorSubcoreMesh(core_axis_name="c", subcore_axis_name="s")

    @functools.partial(
        pl.kernel,
        mesh=mesh,
        compiler_params=pltpu.CompilerParams(needs_layout_passes=False),
        out_type=jax.ShapeDtypeStruct((batch, _NCOL), jnp.float32),
        scratch_types=[
            pltpu.VMEM((rows_w * _NFIELD,), jnp.int32),
            pltpu.VMEM((_CHUNKS * _LANES,), jnp.int32),
            pltpu.VMEM((sub_rows, _NCOL), jnp.float32),
            pltpu.VMEM((sub_rows, _NCOL), jnp.float32),
            pltpu.SemaphoreType.DMA,
            pltpu.SemaphoreType.DMA,
        ],
    )
    def sc_call(data_hbm, combo_hbm, out_hbm, data_v, combo_v,
                buf0, buf1, sem0, sem1):
        wid = lax.axis_index("s") * nc + lax.axis_index("c")
        row0 = wid * rows_w
        pltpu.sync_copy(data_hbm.at[pl.ds(row0 * _NFIELD, rows_w * _NFIELD)],
                        data_v)
        pltpu.sync_copy(combo_hbm, combo_v)

        bufs, sems = (buf0, buf1), (sem0, sem1)

        def compute_sub_block(sb, buf):
            for j in range(_CHUNKS):
                combo_j = combo_v[pl.ds(j * _LANES, _LANES)]
                f_j = combo_j & 0xFFFF
                l_j = combo_j >> 16

                start = _CHUNK_STARTS[j]

                @plsc.parallel_loop(0, sub_rows, unroll=8)
                def row_body(r, f_j=f_j, l_j=l_j, start=start, buf=buf, sb=sb):
                    base = (sb * sub_rows + r) * _NFIELD
                    thr = plsc.load_gather(data_v, [f_j + base])
                    buf[r, pl.ds(start, _LANES)] = jnp.where(
                        l_j <= thr, jnp.float32(1.0), jnp.float32(0.0))

        def make_copy(sb, half):
            dst = out_hbm.at[pl.ds(row0 + sb * sub_rows, sub_rows), :]
            return pltpu.make_async_copy(bufs[half], dst, sems[half])

        for half in (0, 1):
            compute_sub_block(jnp.int32(half), bufs[half])
            make_copy(jnp.int32(half), half).start()

        def pair_body(sbp, carry):
            for half in (0, 1):
                sb = sbp * 2 + half
                make_copy(sb - 2, half).wait()
                compute_sub_block(sb, bufs[half])
                make_copy(sb, half).start()
            return carry

        lax.fori_loop(1, nsb // 2, pair_body, 0)
        make_copy(nsb - 2, 0).wait()
        make_copy(nsb - 1, 1).wait()

    return sc_call


def kernel(data, accum_mat):
    del accum_mat
    batch = data.shape[0]
    combo = jnp.asarray(_COMBO_NP)
    return _make_sc_call(batch)(data.reshape(-1).astype(jnp.int32), combo)
